# Initial kernel scaffold; baseline (speedup 1.0000x reference)
import jax
import jax.numpy as jnp
from jax.experimental import pallas as pl


def kernel(u_ids, pos_ids, neg_ids, feat_v, feat_t, edge_index, pref_v, pref_t, W_v, b_v, W_t, b_t, item_modality_weights):
    raise NotImplementedError("write your pallas kernel here")


# speedup vs baseline: 9.8849x; 9.8849x over previous
import functools

import jax
import jax.numpy as jnp
from jax import lax
from jax.experimental import pallas as pl
from jax.experimental.pallas import tpu as pltpu
from jax.experimental.pallas import tpu_sc as plsc

NU = 20000
NI = 20000
NNODE = NU + NI
NNP = 40064
CW = 16
NCH = 128 // CW
CPC = NCH // 2
ADT = jnp.bfloat16
DQ = NNP // 4
DACC = 10112
DRPT = DACC // 16
DDMP = DQ // 16
E = 640000
EP = 643072
NIT = EP // 16 // 128
NITW = EP // 32 // 128
RPT = NNP // 16
ZB = 626
BATCH = 4096
NG = 3 * BATCH
NGW = NG // 32
WD = 1e-4

_mesh = plsc.VectorSubcoreMesh(core_axis_name="c", subcore_axis_name="s")
_sc_params = pltpu.CompilerParams(use_tc_tiling_on_sc=False)


@functools.partial(
    pl.kernel, mesh=_mesh, compiler_params=_sc_params,
    out_type=jax.ShapeDtypeStruct((4, DQ, 8), jnp.float32),
    scratch_types=[
        pltpu.VMEM((NIT, 128), jnp.int32),
        pltpu.VMEM((1, 128), jnp.int32),
        pltpu.VMEM((128, 8), jnp.float32),
        pltpu.VMEM_SHARED((DACC, 8), jnp.float32),
    ],
)
def _deg_sc(ridx16_hbm, ones_hbm, zeros_hbm, out_hbm, ridx_vm, idx2_vm,
            ones_vm, acc):
    c = lax.axis_index("c")
    s = lax.axis_index("s")
    pltpu.sync_copy(ridx16_hbm.at[s], ridx_vm)
    pltpu.sync_copy(ones_hbm, ones_vm)
    trash = DQ + lax.iota(jnp.int32, 16) * 6

    for k in range(2):
        q = c * 2 + k
        base = q * DQ
        pltpu.sync_copy(zeros_hbm, acc.at[pl.ds(s * DRPT, DRPT)])
        plsc.subcore_barrier()

        def body(j, carry):
            for v in range(8):
                idx = ridx_vm[j, pl.ds(v * 16, 16)]
                local = idx - base
                inb = (local >= 0) & (local < DQ)
                idx2_vm[0, pl.ds(v * 16, 16)] = jnp.where(inb, local, trash)
            pltpu.sync_copy(ones_vm, acc.at[idx2_vm.at[0]], add=True)
            return carry

        lax.fori_loop(0, NIT, body, 0)
        plsc.subcore_barrier()
        pltpu.sync_copy(acc.at[pl.ds(s * DDMP, DDMP)],
                        out_hbm.at[q, pl.ds(s * DDMP, DDMP)])
        plsc.subcore_barrier()


@functools.partial(
    pl.kernel, mesh=_mesh, compiler_params=_sc_params,
    out_type=[
        jax.ShapeDtypeStruct((NCH, NNP, CW), ADT),
        jax.ShapeDtypeStruct((NCH * NNP, CW), ADT),
    ],
    scratch_types=[
        pltpu.VMEM((NIT, 128), jnp.int32),
        pltpu.VMEM((NIT, 128), jnp.int32),
        pltpu.VMEM((128, CW), ADT),
        pltpu.VMEM((ZB, CW), ADT),
        pltpu.VMEM((ZB, CW), ADT),
        pltpu.VMEM_SHARED((NNP, CW), ADT),
        pltpu.SemaphoreType.DMA,
    ],
)
def _gcn_sc(table1_hbm, cidx_hbm, ridx_hbm, invdegb_hbm, zeros_hbm,
            s12_hbm, t2_hbm,
            cidx_vm, ridx_vm, rows_vm, tbuf, ibuf, acc, sem):
    c = lax.axis_index("c")
    s = lax.axis_index("s")
    pltpu.sync_copy(ridx_hbm.at[s], ridx_vm)

    def segsum(table, chunk):
        pltpu.sync_copy(cidx_hbm.at[chunk, s], cidx_vm)

        def body(j, carry):
            pltpu.async_copy(table.at[cidx_vm.at[j]], rows_vm, sem).wait()
            pltpu.sync_copy(rows_vm, acc.at[ridx_vm.at[j]], add=True)
            return carry

        lax.fori_loop(0, NIT, body, 0)
        plsc.subcore_barrier()

    for k in range(CPC):
        chunk = c * CPC + k
        pltpu.sync_copy(zeros_hbm, acc.at[pl.ds(s * RPT, RPT)])
        plsc.subcore_barrier()
        segsum(table1_hbm, chunk)
        for m in range(4):
            base = s * RPT + m * ZB
            pltpu.sync_copy(acc.at[pl.ds(base, ZB)], tbuf)
            pltpu.sync_copy(invdegb_hbm.at[pl.ds(base, ZB)], ibuf)

            def scale(i, carry):
                tbuf[pl.ds(i * 2, 2), :] = (tbuf[pl.ds(i * 2, 2), :]
                                            * ibuf[pl.ds(i * 2, 2), :])
                return carry

            lax.fori_loop(0, ZB // 2, scale, 0)
            pltpu.sync_copy(tbuf, t2_hbm.at[pl.ds(chunk * NNP + base, ZB)])
        plsc.subcore_barrier()
        segsum(t2_hbm, chunk)
        pltpu.sync_copy(acc.at[pl.ds(s * RPT, RPT)],
                        s12_hbm.at[chunk, pl.ds(s * RPT, RPT)])
        plsc.subcore_barrier()


@functools.partial(
    pl.kernel, mesh=_mesh, compiler_params=_sc_params,
    out_type=jax.ShapeDtypeStruct((NG, 128), jnp.float32),
    scratch_types=[
        pltpu.VMEM((NGW // 128, 128), jnp.int32),
        pltpu.VMEM((128, 128), jnp.float32),
        pltpu.SemaphoreType.DMA,
    ],
)
def _gather_sc(table_hbm, gidx_hbm, out_hbm, gidx_vm, rows_vm, sem):
    c = lax.axis_index("c")
    s = lax.axis_index("s")
    w = c * 16 + s
    pltpu.sync_copy(gidx_hbm.at[w], gidx_vm)

    def body(j, carry):
        pltpu.async_copy(table_hbm.at[gidx_vm.at[j]], rows_vm, sem).wait()
        pltpu.sync_copy(rows_vm, out_hbm.at[pl.ds(w * NGW + j * 128, 128)])
        return carry

    lax.fori_loop(0, NGW // 128, body, 0)


def _mm_body(fv_ref, ft_ref, wv_ref, wt_ref, bv_ref, bt_ref, ov, ot):
    ov[...] = jnp.dot(fv_ref[...], wv_ref[...],
                      preferred_element_type=jnp.float32) + bv_ref[...]
    ot[...] = jnp.dot(ft_ref[...], wt_ref[...],
                      preferred_element_type=jnp.float32) + bt_ref[...]


def _item_emb_chunks(feat_v, feat_t, W_v, b_v, W_t, b_t):
    R = 800
    g = NI // R
    return pl.pallas_call(
        _mm_body,
        grid=(g,),
        in_specs=[
            pl.BlockSpec((R, 2048), lambda i: (i, 0)),
            pl.BlockSpec((R, 768), lambda i: (i, 0)),
            pl.BlockSpec((2048, 64), lambda i: (0, 0)),
            pl.BlockSpec((768, 64), lambda i: (0, 0)),
            pl.BlockSpec((1, 64), lambda i: (0, 0)),
            pl.BlockSpec((1, 64), lambda i: (0, 0)),
        ],
        out_specs=[pl.BlockSpec((R, 64), lambda i: (i, 0))] * 2,
        out_shape=[jax.ShapeDtypeStruct((NI, 64), jnp.float32)] * 2,
    )(feat_v, feat_t, W_v, W_t, b_v.reshape(1, 64), b_t.reshape(1, 64))


def _deg_of(degp):
    return degp[:, :1]


def _scale1_body(x_ref, degp_ref, o_ref, inv_ref):
    deg = jnp.maximum(_deg_of(degp_ref[...]), 1.0)
    o_ref[...] = (x_ref[...] * lax.rsqrt(deg)).astype(ADT)
    inv_ref[...] = jnp.broadcast_to(1.0 / deg, inv_ref.shape).astype(ADT)


def _final_body(x_ref, s12_ref, degp_ref, o_ref):
    r = lax.rsqrt(jnp.maximum(_deg_of(degp_ref[...]), 1.0))
    o_ref[...] = (x_ref[...]
                  + s12_ref[...].astype(jnp.float32) * r) * (1.0 / 3.0)


def _scale1(X, degp):
    R = 5008
    g = NNP // R
    row_spec = pl.BlockSpec((R, 128), lambda i: (i, 0))
    deg_spec = pl.BlockSpec((R, 8), lambda i: (i, 0))
    return pl.pallas_call(
        _scale1_body,
        grid=(g,),
        in_specs=[row_spec, deg_spec],
        out_specs=[row_spec, pl.BlockSpec((R, CW), lambda i: (i, 0))],
        out_shape=[jax.ShapeDtypeStruct((NNP, 128), ADT),
                   jax.ShapeDtypeStruct((NNP, CW), ADT)],
    )(X, degp)


def _final(X, s12, degp):
    R = 5008
    g = NNP // R
    row_spec = pl.BlockSpec((R, 128), lambda i: (i, 0))
    deg_spec = pl.BlockSpec((R, 8), lambda i: (i, 0))
    return pl.pallas_call(
        _final_body,
        grid=(g,),
        in_specs=[row_spec, row_spec, deg_spec],
        out_specs=row_spec,
        out_shape=jax.ShapeDtypeStruct((NNP, 128), jnp.float32),
    )(X, s12, degp)


def _loss_body(g_ref, pv_ref, pt_ref, o_ref):
    g = g_ref[...].reshape(3, BATCH, 128)
    u = g[0]
    p = g[1]
    n = g[2]
    x = jnp.sum(u * p - u * n, axis=1)
    ls = jnp.where(x < 0, x, 0.0) - jnp.log(1.0 + jnp.exp(-jnp.abs(x)))
    bpr = -jnp.mean(ls)
    reg = WD * (jnp.sum(pv_ref[...] ** 2) + jnp.sum(pt_ref[...] ** 2)) * 0.5
    o_ref[...] = jnp.reshape(bpr + reg, (1, 1))


def _loss(gath, pref_v, pref_t):
    return pl.pallas_call(
        _loss_body,
        in_specs=[
            pl.BlockSpec((NG, 128), lambda: (0, 0)),
            pl.BlockSpec((NU, 64), lambda: (0, 0)),
            pl.BlockSpec((NU, 64), lambda: (0, 0)),
        ],
        out_specs=pl.BlockSpec((1, 1), lambda: (0, 0)),
        out_shape=jax.ShapeDtypeStruct((1, 1), jnp.float32),
    )(gath, pref_v, pref_t)


def kernel(u_ids, pos_ids, neg_ids, feat_v, feat_t, edge_index, pref_v,
           pref_t, W_v, b_v, W_t, b_t, item_modality_weights):
    i32 = jnp.int32
    eu = edge_index[0].astype(i32)
    ei = edge_index[1].astype(i32)

    npad = EP - E
    pad_r = NNODE + jnp.arange(npad, dtype=i32) % 64
    pad_c = jnp.arange(npad, dtype=i32) % NNODE
    rows = jnp.concatenate([eu, ei + NU, pad_r])
    cols = jnp.concatenate([ei + NU, eu, pad_c])
    ridx16 = rows.reshape(16, NIT, 128)
    ridx32 = rows.reshape(32, NITW, 128)
    cidx = (jnp.arange(NCH, dtype=i32)[:, None] * NNP
            + cols[None, :]).reshape(NCH, 16, NIT, 128)

    ones8 = jnp.ones((128, 8), jnp.float32)
    zeros8 = jnp.zeros((DRPT, 8), jnp.float32)
    zerosC = jnp.zeros((RPT, CW), ADT)

    degp4 = _deg_sc(ridx16, ones8, zeros8)
    degp = degp4.reshape(NNP, 8)

    mv, mt = _item_emb_chunks(feat_v, feat_t, W_v, b_v, W_t, b_t)
    zpad = jnp.zeros((NNP - NNODE, 128), jnp.float32)
    X = jnp.concatenate([
        jnp.concatenate([pref_v, pref_t], axis=1),
        jnp.concatenate([mv, mt], axis=1),
        zpad], axis=0)

    x1t, invdegb = _scale1(X, degp)
    x1t_cm = x1t.reshape(NNP, NCH, CW).transpose(1, 0, 2)
    s12, _t2 = _gcn_sc(x1t_cm.reshape(NCH * NNP, CW), cidx, ridx16,
                       invdegb, zerosC)
    s12_nm = s12.transpose(1, 0, 2).reshape(NNP, 128)
    out = _final(X, s12_nm, degp)

    all_ids = jnp.concatenate(
        [u_ids.astype(i32), pos_ids.astype(i32) + NU,
         neg_ids.astype(i32) + NU])
    gidx = all_ids.reshape(32, NGW // 128, 128)
    gath = _gather_sc(out, gidx)

    loss = _loss(gath, pref_v, pref_t)
    return loss.reshape(())


# speedup vs baseline: 24.3116x; 2.4595x over previous
import functools

import jax
import jax.numpy as jnp
from jax import lax
from jax.experimental import pallas as pl
from jax.experimental.pallas import tpu as pltpu
from jax.experimental.pallas import tpu_sc as plsc

NU = 20000
NI = 20000
NNODE = NU + NI
NNP = 40064
CW = 16
NCH = 128 // CW
CPC = NCH // 2
ADT = jnp.bfloat16
DQ = NNP // 4
DACC = 10112
DRPT = DACC // 16
DDMP = DQ // 16
E = 640000
EP = 655360
NIT = EP // 16 // 128
NBUF = 8
NGRP = NIT // NBUF
RPT = NNP // 16
ZB = 626
BATCH = 4096
NG = 3 * BATCH
NGW = NG // 32
WD = 1e-4

_mesh = plsc.VectorSubcoreMesh(core_axis_name="c", subcore_axis_name="s")
_sc_params = pltpu.CompilerParams(use_tc_tiling_on_sc=False)


@functools.partial(
    pl.kernel, mesh=_mesh, compiler_params=_sc_params,
    out_type=jax.ShapeDtypeStruct((4, DQ, 8), jnp.float32),
    scratch_types=[
        pltpu.VMEM((NIT, 128), jnp.int32),
        pltpu.VMEM((NBUF, 128), jnp.int32),
        pltpu.VMEM((128, 8), jnp.float32),
        pltpu.VMEM_SHARED((DACC, 8), jnp.float32),
    ] + [pltpu.SemaphoreType.DMA] * NBUF,
)
def _deg_sc(ridx16_hbm, ones_hbm, zeros_hbm, out_hbm, ridx_vm, idx2_vm,
            ones_vm, acc, *ssem):
    c = lax.axis_index("c")
    s = lax.axis_index("s")
    pltpu.sync_copy(ridx16_hbm.at[s], ridx_vm)
    pltpu.sync_copy(ones_hbm, ones_vm)
    trash = DQ + lax.iota(jnp.int32, 16) * 6

    for k in range(2):
        q = c * 2 + k
        base = q * DQ
        pltpu.sync_copy(zeros_hbm, acc.at[pl.ds(s * DRPT, DRPT)])
        plsc.subcore_barrier()

        def body(g, carry):
            for b in range(NBUF):
                j = g * NBUF + b

                @pl.when(g > 0)
                def _():
                    pltpu.make_async_copy(
                        ones_vm, acc.at[idx2_vm.at[b]], ssem[b]).wait()

                for v in range(8):
                    idx = ridx_vm[j, pl.ds(v * 16, 16)]
                    local = idx - base
                    inb = (local >= 0) & (local < DQ)
                    idx2_vm[b, pl.ds(v * 16, 16)] = jnp.where(inb, local,
                                                              trash)
                pltpu.async_copy(ones_vm, acc.at[idx2_vm.at[b]], ssem[b],
                                 add=True)
            return carry

        lax.fori_loop(0, NGRP, body, 0)
        for b in range(NBUF):
            pltpu.make_async_copy(ones_vm, acc.at[idx2_vm.at[b]],
                                  ssem[b]).wait()
        plsc.subcore_barrier()
        pltpu.sync_copy(acc.at[pl.ds(s * DDMP, DDMP)],
                        out_hbm.at[q, pl.ds(s * DDMP, DDMP)])
        plsc.subcore_barrier()


@functools.partial(
    pl.kernel, mesh=_mesh, compiler_params=_sc_params,
    out_type=[
        jax.ShapeDtypeStruct((NCH, NNP, CW), ADT),
        jax.ShapeDtypeStruct((NCH * NNP, CW), ADT),
    ],
    scratch_types=[
        pltpu.VMEM((NIT, 128), jnp.int32),
        pltpu.VMEM((NIT, 128), jnp.int32),
        pltpu.VMEM((NBUF, 128, CW), ADT),
        pltpu.VMEM((ZB, CW), ADT),
        pltpu.VMEM((ZB, CW), ADT),
        pltpu.VMEM_SHARED((NNP, CW), ADT),
    ] + [pltpu.SemaphoreType.DMA] * (2 * NBUF),
)
def _gcn_sc(table1_hbm, cidx_hbm, ridx_hbm, invdegb_hbm, zeros_hbm,
            s12_hbm, t2_hbm,
            cidx_vm, ridx_vm, rows_vm, tbuf, ibuf, acc, *sems):
    gsem = sems[:NBUF]
    ssem = sems[NBUF:]
    c = lax.axis_index("c")
    s = lax.axis_index("s")
    pltpu.sync_copy(ridx_hbm.at[s], ridx_vm)

    def segsum(table, chunk):
        pltpu.sync_copy(cidx_hbm.at[chunk, s], cidx_vm)
        for b in range(NBUF):
            pltpu.async_copy(table.at[cidx_vm.at[b]], rows_vm.at[b], gsem[b])

        def body(g, carry):
            for b in range(NBUF):
                j = g * NBUF + b
                pltpu.make_async_copy(table.at[cidx_vm.at[j]],
                                      rows_vm.at[b], gsem[b]).wait()
                pltpu.async_copy(rows_vm.at[b], acc.at[ridx_vm.at[j]],
                                 ssem[b], add=True)
            for b in range(NBUF):
                j = g * NBUF + b
                pltpu.make_async_copy(rows_vm.at[b], acc.at[ridx_vm.at[j]],
                                      ssem[b]).wait()
                pltpu.async_copy(table.at[cidx_vm.at[j + NBUF]],
                                 rows_vm.at[b], gsem[b])
            return carry

        lax.fori_loop(0, NGRP - 1, body, 0)
        for b in range(NBUF):
            j = (NGRP - 1) * NBUF + b
            pltpu.make_async_copy(table.at[cidx_vm.at[j]],
                                  rows_vm.at[b], gsem[b]).wait()
            pltpu.async_copy(rows_vm.at[b], acc.at[ridx_vm.at[j]],
                             ssem[b], add=True)
        for b in range(NBUF):
            j = (NGRP - 1) * NBUF + b
            pltpu.make_async_copy(rows_vm.at[b], acc.at[ridx_vm.at[j]],
                                  ssem[b]).wait()
        plsc.subcore_barrier()

    for k in range(CPC):
        chunk = c * CPC + k
        pltpu.sync_copy(zeros_hbm, acc.at[pl.ds(s * RPT, RPT)])
        plsc.subcore_barrier()
        segsum(table1_hbm, chunk)
        for m in range(4):
            base = s * RPT + m * ZB
            pltpu.sync_copy(acc.at[pl.ds(base, ZB)], tbuf)
            pltpu.sync_copy(invdegb_hbm.at[pl.ds(base, ZB)], ibuf)

            def scale(i, carry):
                tbuf[pl.ds(i * 2, 2), :] = (tbuf[pl.ds(i * 2, 2), :]
                                            * ibuf[pl.ds(i * 2, 2), :])
                return carry

            lax.fori_loop(0, ZB // 2, scale, 0)
            pltpu.sync_copy(tbuf, t2_hbm.at[pl.ds(chunk * NNP + base, ZB)])
        plsc.subcore_barrier()
        segsum(t2_hbm, chunk)
        pltpu.sync_copy(acc.at[pl.ds(s * RPT, RPT)],
                        s12_hbm.at[chunk, pl.ds(s * RPT, RPT)])
        plsc.subcore_barrier()


@functools.partial(
    pl.kernel, mesh=_mesh, compiler_params=_sc_params,
    out_type=jax.ShapeDtypeStruct((NG, 128), jnp.float32),
    scratch_types=[
        pltpu.VMEM((NGW // 128, 128), jnp.int32),
        pltpu.VMEM((128, 128), jnp.float32),
        pltpu.SemaphoreType.DMA,
    ],
)
def _gather_sc(table_hbm, gidx_hbm, out_hbm, gidx_vm, rows_vm, sem):
    c = lax.axis_index("c")
    s = lax.axis_index("s")
    w = c * 16 + s
    pltpu.sync_copy(gidx_hbm.at[w], gidx_vm)

    def body(j, carry):
        pltpu.async_copy(table_hbm.at[gidx_vm.at[j]], rows_vm, sem).wait()
        pltpu.sync_copy(rows_vm, out_hbm.at[pl.ds(w * NGW + j * 128, 128)])
        return carry

    lax.fori_loop(0, NGW // 128, body, 0)


def _mm_body(fv_ref, ft_ref, wv_ref, wt_ref, bv_ref, bt_ref, ov, ot):
    ov[...] = jnp.dot(fv_ref[...], wv_ref[...],
                      preferred_element_type=jnp.float32) + bv_ref[...]
    ot[...] = jnp.dot(ft_ref[...], wt_ref[...],
                      preferred_element_type=jnp.float32) + bt_ref[...]


def _item_emb_chunks(feat_v, feat_t, W_v, b_v, W_t, b_t):
    R = 800
    g = NI // R
    return pl.pallas_call(
        _mm_body,
        grid=(g,),
        in_specs=[
            pl.BlockSpec((R, 2048), lambda i: (i, 0)),
            pl.BlockSpec((R, 768), lambda i: (i, 0)),
            pl.BlockSpec((2048, 64), lambda i: (0, 0)),
            pl.BlockSpec((768, 64), lambda i: (0, 0)),
            pl.BlockSpec((1, 64), lambda i: (0, 0)),
            pl.BlockSpec((1, 64), lambda i: (0, 0)),
        ],
        out_specs=[pl.BlockSpec((R, 64), lambda i: (i, 0))] * 2,
        out_shape=[jax.ShapeDtypeStruct((NI, 64), jnp.float32)] * 2,
    )(feat_v, feat_t, W_v, W_t, b_v.reshape(1, 64), b_t.reshape(1, 64))


def _deg_of(degp):
    return degp[:, :1]


def _scale1_body(x_ref, degp_ref, o_ref, inv_ref):
    deg = jnp.maximum(_deg_of(degp_ref[...]), 1.0)
    o_ref[...] = (x_ref[...] * lax.rsqrt(deg)).astype(ADT)
    inv_ref[...] = jnp.broadcast_to(1.0 / deg, inv_ref.shape).astype(ADT)


def _final_body(x_ref, s12_ref, degp_ref, o_ref):
    r = lax.rsqrt(jnp.maximum(_deg_of(degp_ref[...]), 1.0))
    o_ref[...] = (x_ref[...]
                  + s12_ref[...].astype(jnp.float32) * r) * (1.0 / 3.0)


def _scale1(X, degp):
    R = 5008
    g = NNP // R
    row_spec = pl.BlockSpec((R, 128), lambda i: (i, 0))
    deg_spec = pl.BlockSpec((R, 8), lambda i: (i, 0))
    return pl.pallas_call(
        _scale1_body,
        grid=(g,),
        in_specs=[row_spec, deg_spec],
        out_specs=[row_spec, pl.BlockSpec((R, CW), lambda i: (i, 0))],
        out_shape=[jax.ShapeDtypeStruct((NNP, 128), ADT),
                   jax.ShapeDtypeStruct((NNP, CW), ADT)],
    )(X, degp)


def _final(X, s12, degp):
    R = 5008
    g = NNP // R
    row_spec = pl.BlockSpec((R, 128), lambda i: (i, 0))
    deg_spec = pl.BlockSpec((R, 8), lambda i: (i, 0))
    return pl.pallas_call(
        _final_body,
        grid=(g,),
        in_specs=[row_spec, row_spec, deg_spec],
        out_specs=row_spec,
        out_shape=jax.ShapeDtypeStruct((NNP, 128), jnp.float32),
    )(X, s12, degp)


def _loss_body(g_ref, pv_ref, pt_ref, o_ref):
    g = g_ref[...].reshape(3, BATCH, 128)
    u = g[0]
    p = g[1]
    n = g[2]
    x = jnp.sum(u * p - u * n, axis=1)
    ls = jnp.where(x < 0, x, 0.0) - jnp.log(1.0 + jnp.exp(-jnp.abs(x)))
    bpr = -jnp.mean(ls)
    reg = WD * (jnp.sum(pv_ref[...] ** 2) + jnp.sum(pt_ref[...] ** 2)) * 0.5
    o_ref[...] = jnp.reshape(bpr + reg, (1, 1))


def _loss(gath, pref_v, pref_t):
    return pl.pallas_call(
        _loss_body,
        in_specs=[
            pl.BlockSpec((NG, 128), lambda: (0, 0)),
            pl.BlockSpec((NU, 64), lambda: (0, 0)),
            pl.BlockSpec((NU, 64), lambda: (0, 0)),
        ],
        out_specs=pl.BlockSpec((1, 1), lambda: (0, 0)),
        out_shape=jax.ShapeDtypeStruct((1, 1), jnp.float32),
    )(gath, pref_v, pref_t)


def kernel(u_ids, pos_ids, neg_ids, feat_v, feat_t, edge_index, pref_v,
           pref_t, W_v, b_v, W_t, b_t, item_modality_weights):
    i32 = jnp.int32
    eu = edge_index[0].astype(i32)
    ei = edge_index[1].astype(i32)

    npad = EP - E
    pad_r = NNODE + jnp.arange(npad, dtype=i32) % 64
    pad_c = jnp.arange(npad, dtype=i32) % NNODE
    rows = jnp.concatenate([eu, ei + NU, pad_r])
    cols = jnp.concatenate([ei + NU, eu, pad_c])
    ridx16 = rows.reshape(16, NIT, 128)
    cidx = (jnp.arange(NCH, dtype=i32)[:, None] * NNP
            + cols[None, :]).reshape(NCH, 16, NIT, 128)

    ones8 = jnp.ones((128, 8), jnp.float32)
    zeros8 = jnp.zeros((DRPT, 8), jnp.float32)
    zerosC = jnp.zeros((RPT, CW), ADT)

    degp4 = _deg_sc(ridx16, ones8, zeros8)
    degp = degp4.reshape(NNP, 8)

    mv, mt = _item_emb_chunks(feat_v, feat_t, W_v, b_v, W_t, b_t)
    zpad = jnp.zeros((NNP - NNODE, 128), jnp.float32)
    X = jnp.concatenate([
        jnp.concatenate([pref_v, pref_t], axis=1),
        jnp.concatenate([mv, mt], axis=1),
        zpad], axis=0)

    x1t, invdegb = _scale1(X, degp)
    x1t_cm = x1t.reshape(NNP, NCH, CW).transpose(1, 0, 2)
    s12, _t2 = _gcn_sc(x1t_cm.reshape(NCH * NNP, CW), cidx, ridx16,
                       invdegb, zerosC)
    s12_nm = s12.transpose(1, 0, 2).reshape(NNP, 128)
    out = _final(X, s12_nm, degp)

    all_ids = jnp.concatenate(
        [u_ids.astype(i32), pos_ids.astype(i32) + NU,
         neg_ids.astype(i32) + NU])
    gidx = all_ids.reshape(32, NGW // 128, 128)
    gath = _gather_sc(out, gidx)

    loss = _loss(gath, pref_v, pref_t)
    return loss.reshape(())


# speedup vs baseline: 28.7745x; 1.1836x over previous
import functools

import jax
import jax.numpy as jnp
from jax import lax
from jax.experimental import pallas as pl
from jax.experimental.pallas import tpu as pltpu
from jax.experimental.pallas import tpu_sc as plsc

NU = 20000
NI = 20000
NNODE = NU + NI
NNP = 40064
CW = 16
NCH = 128 // CW
CPC = NCH // 2
DQ = NNP // 4
DACC = 10112
DRPT = DACC // 16
DDMP = DQ // 16
E = 640000
EP = 655360
NIT = EP // 16 // 128
NBUF = 8
NGRP = NIT // NBUF
RPT = NNP // 16
ZB = 626
BATCH = 4096
NG = 3 * BATCH
NGW = NG // 32
WD = 1e-4

_mesh = plsc.VectorSubcoreMesh(core_axis_name="c", subcore_axis_name="s")
_sc_params = pltpu.CompilerParams(use_tc_tiling_on_sc=False)


@functools.partial(
    pl.kernel, mesh=_mesh, compiler_params=_sc_params,
    out_type=jax.ShapeDtypeStruct((4, DQ, 16), jnp.float32),
    scratch_types=[
        pltpu.VMEM((NIT, 128), jnp.int32),
        pltpu.VMEM((NBUF, 128), jnp.int32),
        pltpu.VMEM((128, 8), jnp.float32),
        pltpu.VMEM_SHARED((DACC, 8), jnp.float32),
    ] + [pltpu.SemaphoreType.DMA] * NBUF,
)
def _deg_sc(ridx16_hbm, ones_hbm, zeros_hbm, out_hbm, ridx_vm, idx2_vm,
            ones_vm, acc, *ssem):
    c = lax.axis_index("c")
    s = lax.axis_index("s")
    pltpu.sync_copy(ridx16_hbm.at[s], ridx_vm)
    pltpu.sync_copy(ones_hbm, ones_vm)
    trash = DQ + lax.iota(jnp.int32, 16) * 6

    for k in range(2):
        q = c * 2 + k
        base = q * DQ
        pltpu.sync_copy(zeros_hbm, acc.at[pl.ds(s * DRPT, DRPT)])
        plsc.subcore_barrier()

        def body(g, carry):
            for b in range(NBUF):
                j = g * NBUF + b

                @pl.when(g > 0)
                def _():
                    pltpu.make_async_copy(
                        ones_vm, acc.at[idx2_vm.at[b]], ssem[b]).wait()

                for v in range(8):
                    idx = ridx_vm[j, pl.ds(v * 16, 16)]
                    local = idx - base
                    inb = (local >= 0) & (local < DQ)
                    idx2_vm[b, pl.ds(v * 16, 16)] = jnp.where(inb, local,
                                                              trash)
                pltpu.async_copy(ones_vm, acc.at[idx2_vm.at[b]], ssem[b],
                                 add=True)
            return carry

        lax.fori_loop(0, NGRP, body, 0)
        for b in range(NBUF):
            pltpu.make_async_copy(ones_vm, acc.at[idx2_vm.at[b]],
                                  ssem[b]).wait()
        plsc.subcore_barrier()
        pltpu.sync_copy(acc.at[pl.ds(s * DDMP, DDMP)],
                        out_hbm.at[q, pl.ds(s * DDMP, DDMP), pl.ds(0, 8)])
        pltpu.sync_copy(acc.at[pl.ds(s * DDMP, DDMP)],
                        out_hbm.at[q, pl.ds(s * DDMP, DDMP), pl.ds(8, 8)])
        plsc.subcore_barrier()


def _rtab_body(d_ref, rb_ref, inv_ref):
    deg = jnp.maximum(d_ref[...], 1.0)
    rb_ref[...] = lax.rsqrt(deg)
    inv_ref[...] = 1.0 / deg


def _rtabs(degf):
    n = NNP * 16 // 128
    spec = pl.BlockSpec((n, 128), lambda: (0, 0))
    return pl.pallas_call(
        _rtab_body,
        in_specs=[spec],
        out_specs=[spec, spec],
        out_shape=[jax.ShapeDtypeStruct((n, 128), jnp.float32)] * 2,
    )(degf)


@functools.partial(
    pl.kernel, mesh=_mesh, compiler_params=_sc_params,
    out_type=[
        jax.ShapeDtypeStruct((NCH, NNP, CW), jnp.float32),
        jax.ShapeDtypeStruct((NCH * NNP, CW), jnp.bfloat16),
        jax.ShapeDtypeStruct((NCH * NNP, CW), jnp.bfloat16),
    ],
    scratch_types=[
        pltpu.VMEM((2, NBUF, 128), jnp.int32),
        pltpu.VMEM((2, NBUF, 128), jnp.int32),
        pltpu.VMEM((NBUF, 128, CW), jnp.bfloat16),
        pltpu.VMEM((ZB, 64), jnp.float32),
        pltpu.VMEM((ZB, CW), jnp.float32),
        pltpu.VMEM((ZB, CW), jnp.bfloat16),
        pltpu.VMEM((ZB, CW), jnp.float32),
        pltpu.VMEM_SHARED((NNP, CW), jnp.bfloat16),
    ] + [pltpu.SemaphoreType.DMA] * (2 * NBUF + 2),
)
def _gcn_sc(x_hbm, rb_hbm, inv_hbm, cidx_hbm, ridx_hbm, zeros_hbm,
            s12_hbm, x1t_hbm, t2_hbm,
            cidx_vm, ridx_vm, rows_vm, xbuf, fbuf, bbuf, obuf, acc, *sems):
    gsem = sems[:NBUF]
    ssem = sems[NBUF:2 * NBUF]
    isem = sems[2 * NBUF]
    jsem = sems[2 * NBUF + 1]
    c = lax.axis_index("c")
    s = lax.axis_index("s")

    for cc in range(2):

        @pl.when(c == cc)
        def _():
            for m in range(4):
                base = s * RPT + m * ZB
                pltpu.sync_copy(
                    x_hbm.at[pl.ds(base, ZB), pl.ds(cc * 64, 64)], xbuf)
                pltpu.sync_copy(rb_hbm.at[pl.ds(base, ZB)], fbuf)
                for k in range(CPC):
                    ch = cc * CPC + k
                    lo = k * 16

                    def xbody(i, carry):
                        x2 = xbuf[pl.ds(i * 2, 2), pl.ds(lo, 16)]
                        r2 = fbuf[pl.ds(i * 2, 2), :]
                        bbuf[pl.ds(i * 2, 2), :] = (
                            x2 * r2).astype(jnp.bfloat16)
                        return carry

                    lax.fori_loop(0, ZB // 2, xbody, 0)
                    pltpu.sync_copy(
                        bbuf, x1t_hbm.at[pl.ds(ch * NNP + base, ZB)])
    plsc.subcore_barrier()

    def segsum(table, chunk):
        def stage(g, p):
            pltpu.async_copy(cidx_hbm.at[chunk, s, pl.ds(g * NBUF, NBUF)],
                             cidx_vm.at[p], isem)
            pltpu.async_copy(ridx_hbm.at[s, pl.ds(g * NBUF, NBUF)],
                             ridx_vm.at[p], jsem)

        def stage_wait(g, p):
            pltpu.make_async_copy(
                cidx_hbm.at[chunk, s, pl.ds(g * NBUF, NBUF)],
                cidx_vm.at[p], isem).wait()
            pltpu.make_async_copy(
                ridx_hbm.at[s, pl.ds(g * NBUF, NBUF)],
                ridx_vm.at[p], jsem).wait()

        stage(0, 0)
        stage_wait(0, 0)
        stage(1, 1)
        stage_wait(1, 1)
        for b in range(NBUF):
            pltpu.async_copy(table.at[cidx_vm.at[0, b]], rows_vm.at[b],
                             gsem[b])

        def body(g, carry):
            p = g % 2
            pn = (g + 1) % 2
            for b in range(NBUF):
                pltpu.make_async_copy(table.at[cidx_vm.at[p, b]],
                                      rows_vm.at[b], gsem[b]).wait()
                pltpu.async_copy(rows_vm.at[b], acc.at[ridx_vm.at[p, b]],
                                 ssem[b], add=True)

            @pl.when(g > 0)
            def _():
                stage_wait(g + 1, pn)

            for b in range(NBUF):
                pltpu.make_async_copy(rows_vm.at[b], acc.at[ridx_vm.at[p, b]],
                                      ssem[b]).wait()
                pltpu.async_copy(table.at[cidx_vm.at[pn, b]],
                                 rows_vm.at[b], gsem[b])

            @pl.when(g + 2 < NGRP)
            def _():
                stage(g + 2, p)
            return carry

        lax.fori_loop(0, NGRP - 1, body, 0)
        p = (NGRP - 1) % 2
        for b in range(NBUF):
            pltpu.make_async_copy(table.at[cidx_vm.at[p, b]],
                                  rows_vm.at[b], gsem[b]).wait()
            pltpu.async_copy(rows_vm.at[b], acc.at[ridx_vm.at[p, b]],
                             ssem[b], add=True)
        for b in range(NBUF):
            pltpu.make_async_copy(rows_vm.at[b], acc.at[ridx_vm.at[p, b]],
                                  ssem[b]).wait()
        plsc.subcore_barrier()

    for k in range(CPC):
        chunk = c * CPC + k
        pltpu.sync_copy(zeros_hbm, acc.at[pl.ds(s * RPT, RPT)])
        plsc.subcore_barrier()
        segsum(x1t_hbm, chunk)
        for m in range(4):
            base = s * RPT + m * ZB
            pltpu.sync_copy(acc.at[pl.ds(base, ZB)], bbuf)
            pltpu.sync_copy(inv_hbm.at[pl.ds(base, ZB)], fbuf)

            def scale(i, carry):
                v = bbuf[pl.ds(i * 2, 2), :].astype(jnp.float32)
                bbuf[pl.ds(i * 2, 2), :] = (
                    v * fbuf[pl.ds(i * 2, 2), :]).astype(jnp.bfloat16)
                return carry

            lax.fori_loop(0, ZB // 2, scale, 0)
            pltpu.sync_copy(bbuf, t2_hbm.at[pl.ds(chunk * NNP + base, ZB)])
        plsc.subcore_barrier()
        segsum(t2_hbm, chunk)
        for m in range(4):
            base = s * RPT + m * ZB
            pltpu.sync_copy(acc.at[pl.ds(base, ZB)], bbuf)

            def cvt(i, carry):
                obuf[pl.ds(i * 2, 2), :] = (
                    bbuf[pl.ds(i * 2, 2), :].astype(jnp.float32))
                return carry

            lax.fori_loop(0, ZB // 2, cvt, 0)
            pltpu.sync_copy(obuf, s12_hbm.at[chunk, pl.ds(base, ZB)])
        plsc.subcore_barrier()


@functools.partial(
    pl.kernel, mesh=_mesh, compiler_params=_sc_params,
    out_type=[
        jax.ShapeDtypeStruct((NG, 128), jnp.float32),
        jax.ShapeDtypeStruct((NG * NCH, CW), jnp.float32),
        jax.ShapeDtypeStruct((NG, 16), jnp.float32),
    ],
    scratch_types=[
        pltpu.VMEM((NGW // 128, 128), jnp.int32),
        pltpu.VMEM((NGW * NCH // 128, 128), jnp.int32),
        pltpu.VMEM((128, 128), jnp.float32),
        pltpu.VMEM((128, CW), jnp.float32),
        pltpu.VMEM((128, 16), jnp.float32),
        pltpu.SemaphoreType.DMA,
    ],
)
def _gather_sc(x_hbm, s12_hbm, deg_hbm, gidx_hbm, gsidx_hbm,
               gx_hbm, gs_hbm, gd_hbm, gidx_vm, gsidx_vm,
               xrow_vm, srow_vm, drow_vm, sem):
    c = lax.axis_index("c")
    s = lax.axis_index("s")
    w = c * 16 + s
    pltpu.sync_copy(gidx_hbm.at[w], gidx_vm)
    pltpu.sync_copy(gsidx_hbm.at[w], gsidx_vm)

    def bx(j, carry):
        pltpu.async_copy(x_hbm.at[gidx_vm.at[j]], xrow_vm, sem).wait()
        pltpu.sync_copy(xrow_vm, gx_hbm.at[pl.ds(w * NGW + j * 128, 128)])
        pltpu.async_copy(deg_hbm.at[gidx_vm.at[j]], drow_vm, sem).wait()
        pltpu.sync_copy(drow_vm, gd_hbm.at[pl.ds(w * NGW + j * 128, 128)])
        return carry

    lax.fori_loop(0, NGW // 128, bx, 0)

    def bs(j, carry):
        pltpu.async_copy(s12_hbm.at[gsidx_vm.at[j]], srow_vm, sem).wait()
        pltpu.sync_copy(srow_vm,
                        gs_hbm.at[pl.ds(w * NGW * NCH + j * 128, 128)])
        return carry

    lax.fori_loop(0, NGW * NCH // 128, bs, 0)


def _mm_body(fv_ref, ft_ref, wv_ref, wt_ref, bv_ref, bt_ref, ov, ot):
    ov[...] = jnp.dot(fv_ref[...], wv_ref[...],
                      preferred_element_type=jnp.float32) + bv_ref[...]
    ot[...] = jnp.dot(ft_ref[...], wt_ref[...],
                      preferred_element_type=jnp.float32) + bt_ref[...]


def _item_emb(feat_v, feat_t, W_v, b_v, W_t, b_t):
    R = 800
    g = NI // R
    return pl.pallas_call(
        _mm_body,
        grid=(g,),
        in_specs=[
            pl.BlockSpec((R, 2048), lambda i: (i, 0)),
            pl.BlockSpec((R, 768), lambda i: (i, 0)),
            pl.BlockSpec((2048, 64), lambda i: (0, 0)),
            pl.BlockSpec((768, 64), lambda i: (0, 0)),
            pl.BlockSpec((1, 64), lambda i: (0, 0)),
            pl.BlockSpec((1, 64), lambda i: (0, 0)),
        ],
        out_specs=[pl.BlockSpec((R, 64), lambda i: (i, 0))] * 2,
        out_shape=[jax.ShapeDtypeStruct((NI, 64), jnp.float32)] * 2,
    )(feat_v, feat_t, W_v, W_t, b_v.reshape(1, 64), b_t.reshape(1, 64))


def _loss_body(gx_ref, gs_ref, gd_ref, pv_ref, pt_ref, o_ref):
    r = lax.rsqrt(jnp.maximum(gd_ref[:, :1], 1.0))
    out = (gx_ref[...] + gs_ref[...] * r) * (1.0 / 3.0)
    g = out.reshape(3, BATCH, 128)
    x = jnp.sum(g[0] * g[1] - g[0] * g[2], axis=1)
    ls = jnp.where(x < 0, x, 0.0) - jnp.log(1.0 + jnp.exp(-jnp.abs(x)))
    bpr = -jnp.mean(ls)
    reg = WD * (jnp.sum(pv_ref[...] ** 2) + jnp.sum(pt_ref[...] ** 2)) * 0.5
    o_ref[...] = jnp.reshape(bpr + reg, (1, 1))


def _loss(gx, gs, gd, pref_v, pref_t):
    return pl.pallas_call(
        _loss_body,
        in_specs=[
            pl.BlockSpec((NG, 128), lambda: (0, 0)),
            pl.BlockSpec((NG, 128), lambda: (0, 0)),
            pl.BlockSpec((NG, 16), lambda: (0, 0)),
            pl.BlockSpec((NU, 64), lambda: (0, 0)),
            pl.BlockSpec((NU, 64), lambda: (0, 0)),
        ],
        out_specs=pl.BlockSpec((1, 1), lambda: (0, 0)),
        out_shape=jax.ShapeDtypeStruct((1, 1), jnp.float32),
    )(gx, gs, gd, pref_v, pref_t)


def kernel(u_ids, pos_ids, neg_ids, feat_v, feat_t, edge_index, pref_v,
           pref_t, W_v, b_v, W_t, b_t, item_modality_weights):
    i32 = jnp.int32
    eu = edge_index[0].astype(i32)
    ei = edge_index[1].astype(i32)

    npad = EP - E
    pad_r = NNODE + jnp.arange(npad, dtype=i32) % 64
    pad_c = jnp.arange(npad, dtype=i32) % NNODE
    rows = jnp.concatenate([eu, ei + NU, pad_r])
    cols = jnp.concatenate([ei + NU, eu, pad_c])
    ridx16 = rows.reshape(16, NIT, 128)
    cidx = (jnp.arange(NCH, dtype=i32)[:, None] * NNP
            + cols[None, :]).reshape(NCH, 16, NIT, 128)

    ones16 = jnp.ones((128, 8), jnp.float32)
    zeros16 = jnp.zeros((DRPT, 8), jnp.float32)
    zerosC = jnp.zeros((RPT, CW), jnp.bfloat16)

    degp4 = _deg_sc(ridx16, ones16, zeros16)
    degf = degp4.reshape(NNP * 16 // 128, 128)
    rbf, invf = _rtabs(degf)
    rb = rbf.reshape(NNP, 16)
    invdegb = invf.reshape(NNP, 16)
    deg16 = degp4.reshape(NNP, 16)

    mv, mt = _item_emb(feat_v, feat_t, W_v, b_v, W_t, b_t)
    zpad = jnp.zeros((NNP - NNODE, 128), jnp.float32)
    X = jnp.concatenate([
        jnp.concatenate([pref_v, pref_t], axis=1),
        jnp.concatenate([mv, mt], axis=1),
        zpad], axis=0)

    s12, _x1t, _t2 = _gcn_sc(X, rb, invdegb, cidx, ridx16, zerosC)

    all_ids = jnp.concatenate(
        [u_ids.astype(i32), pos_ids.astype(i32) + NU,
         neg_ids.astype(i32) + NU])
    gidx = all_ids.reshape(32, NGW // 128, 128)
    gsidx = (jnp.arange(NCH, dtype=i32)[None, :] * NNP
             + all_ids[:, None]).reshape(32, NGW * NCH // 128, 128)
    gx, gs, gd = _gather_sc(X, s12.reshape(NCH * NNP, CW), deg16,
                            gidx, gsidx)

    loss = _loss(gx, gs.reshape(NG, 128), gd, pref_v, pref_t)
    return loss.reshape(())
